# baseline (speedup 1.0000x reference)
import functools

import jax
import jax.numpy as jnp
from jax.experimental import pallas as pl
from jax.experimental.pallas import tpu as pltpu
from jax.sharding import PartitionSpec as P

from jax.experimental.shard_map import shard_map

_MASK_IDX = 5
_BM = 2048


def _patch_mm(x_ref, w_ref, b_ref, acts_ref, o_ref, *, blocks_per_batch):
    wb = w_ref[...].astype(jnp.bfloat16)
    y = jnp.dot(x_ref[...].astype(jnp.bfloat16), wb,
                preferred_element_type=jnp.float32)
    y = jnp.dot(y.astype(jnp.bfloat16), wb,
                preferred_element_type=jnp.float32)
    o_ref[...] = y + b_ref[...]

    @pl.when(pl.program_id(0) % blocks_per_batch == 0)
    def _():
        o_ref[_MASK_IDX, :] = acts_ref[0]


def _local(x, W, b2, acts2):
    Bl, S, D = x.shape
    xf = x.reshape(Bl * S, D)
    bm = _BM
    grid = (Bl * S // bm,)
    out = pl.pallas_call(
        functools.partial(_patch_mm, blocks_per_batch=S // bm),
        grid=grid,
        in_specs=[
            pl.BlockSpec((bm, D), lambda i: (i, 0)),
            pl.BlockSpec((D, D), lambda i: (0, 0)),
            pl.BlockSpec((1, D), lambda i: (0, 0)),
            pl.BlockSpec((1, D), lambda i: (0, 0)),
        ],
        out_specs=pl.BlockSpec((bm, D), lambda i: (i, 0)),
        out_shape=jax.ShapeDtypeStruct((Bl * S, D), jnp.float32),
        compiler_params=pltpu.CompilerParams(
            dimension_semantics=("arbitrary",),
        ),
    )(xf, W, b2, acts2)
    return out.reshape(Bl, S, D)


def kernel(x, W, b, acts):
    B, S, D = x.shape
    b2 = b.reshape(1, D)
    acts2 = acts.reshape(1, D)
    return _local(x, W, b2, acts2)


# speedup vs baseline: 1.1169x; 1.1169x over previous
import functools

import jax
import jax.numpy as jnp
from jax.experimental import pallas as pl
from jax.experimental.pallas import tpu as pltpu

_MASK_IDX = 5
_BM = 2048
_BN = 512


def _patch_mm(x_ref, w_ref, b_ref, acts_ref, o_ref, *, blocks_per_batch):
    n = pl.program_id(1)
    xb = x_ref[...].astype(jnp.bfloat16)

    @pl.when(n == 0)
    def _():
        y = jnp.dot(xb, w_ref[:, :_BN].astype(jnp.bfloat16),
                    preferred_element_type=jnp.float32)
        o_ref[...] = y + b_ref[...]

    @pl.when(n == 1)
    def _():
        y = jnp.dot(xb, w_ref[:, _BN:].astype(jnp.bfloat16),
                    preferred_element_type=jnp.float32)
        o_ref[...] = y + b_ref[...]

    @pl.when(pl.program_id(0) % blocks_per_batch == 0)
    def _():
        o_ref[_MASK_IDX, :] = acts_ref[0]


def kernel(x, W, b, acts):
    B, S, D = x.shape
    xf = x.reshape(B * S, D)
    b2 = b.reshape(1, D)
    acts2 = acts.reshape(1, D)
    bm, bn = _BM, _BN
    grid = (B * S // bm, D // bn)
    out = pl.pallas_call(
        functools.partial(_patch_mm, blocks_per_batch=S // bm),
        grid=grid,
        in_specs=[
            pl.BlockSpec((bm, D), lambda i, n: (i, 0)),
            pl.BlockSpec((D, D), lambda i, n: (0, 0)),
            pl.BlockSpec((1, bn), lambda i, n: (0, n)),
            pl.BlockSpec((1, bn), lambda i, n: (0, n)),
        ],
        out_specs=pl.BlockSpec((bm, bn), lambda i, n: (i, n)),
        out_shape=jax.ShapeDtypeStruct((B * S, D), jnp.float32),
        compiler_params=pltpu.CompilerParams(
            dimension_semantics=("arbitrary", "arbitrary"),
        ),
    )(xf, W, b2, acts2)
    return out.reshape(B, S, D)


# speedup vs baseline: 1.5159x; 1.3573x over previous
import functools

import jax
import jax.numpy as jnp
from jax.experimental import pallas as pl
from jax.experimental.pallas import tpu as pltpu

_MASK_IDX = 5
_BM = 2048


def _patch_mm(x_ref, w_ref, b_ref, acts_ref, o_ref, w16_ref, *, blocks_per_batch):
    i = pl.program_id(0)

    @pl.when(i == 0)
    def _():
        w16_ref[...] = w_ref[...].astype(jnp.bfloat16)

    y = jnp.dot(
        x_ref[...].astype(jnp.bfloat16),
        w16_ref[...],
        preferred_element_type=jnp.float32,
    )
    o_ref[...] = y + b_ref[...]

    @pl.when(i % blocks_per_batch == 0)
    def _():
        o_ref[_MASK_IDX, :] = acts_ref[0]


def kernel(x, W, b, acts):
    B, S, D = x.shape
    xf = x.reshape(B * S, D)
    b2 = b.reshape(1, D)
    acts2 = acts.reshape(1, D)
    bm = _BM
    grid = (B * S // bm,)
    out = pl.pallas_call(
        functools.partial(_patch_mm, blocks_per_batch=S // bm),
        grid=grid,
        in_specs=[
            pl.BlockSpec((bm, D), lambda i: (i, 0)),
            pl.BlockSpec((D, D), lambda i: (0, 0)),
            pl.BlockSpec((1, D), lambda i: (0, 0)),
            pl.BlockSpec((1, D), lambda i: (0, 0)),
        ],
        out_specs=pl.BlockSpec((bm, D), lambda i: (i, 0)),
        out_shape=jax.ShapeDtypeStruct((B * S, D), jnp.float32),
        scratch_shapes=[pltpu.VMEM((D, D), jnp.bfloat16)],
        compiler_params=pltpu.CompilerParams(
            dimension_semantics=("arbitrary",),
        ),
    )(xf, W, b2, acts2)
    return out.reshape(B, S, D)


# speedup vs baseline: 1.5236x; 1.0051x over previous
import functools

import jax
import jax.numpy as jnp
from jax.experimental import pallas as pl
from jax.experimental.pallas import tpu as pltpu

_MASK_IDX = 5
_BM = 2048


def _patch_mm(x_ref, w_ref, b_ref, acts_ref, o_ref, *, blocks_per_batch):
    y = jnp.dot(
        x_ref[...].astype(jnp.bfloat16),
        w_ref[...].astype(jnp.bfloat16),
        preferred_element_type=jnp.float32,
    )
    o_ref[...] = y + b_ref[...]

    @pl.when(pl.program_id(0) % blocks_per_batch == 0)
    def _():
        o_ref[_MASK_IDX, :] = acts_ref[0]


def kernel(x, W, b, acts):
    B, S, D = x.shape
    xf = x.reshape(B * S, D)
    b2 = b.reshape(1, D)
    acts2 = acts.reshape(1, D)
    bm = _BM
    grid = (B * S // bm,)
    out = pl.pallas_call(
        functools.partial(_patch_mm, blocks_per_batch=S // bm),
        grid=grid,
        in_specs=[
            pl.BlockSpec((bm, D), lambda i: (i, 0)),
            pl.BlockSpec((D, D), lambda i: (0, 0)),
            pl.BlockSpec((1, D), lambda i: (0, 0)),
            pl.BlockSpec((1, D), lambda i: (0, 0)),
        ],
        out_specs=pl.BlockSpec((bm, D), lambda i: (i, 0)),
        out_shape=jax.ShapeDtypeStruct((B * S, D), jnp.float32),
        compiler_params=pltpu.CompilerParams(
            dimension_semantics=("parallel",),
        ),
    )(xf, W, b2, acts2)
    return out.reshape(B, S, D)


# speedup vs baseline: 1.5431x; 1.0128x over previous
import functools

import jax
import jax.numpy as jnp
from jax.experimental import pallas as pl
from jax.experimental.pallas import tpu as pltpu

_MASK_IDX = 5
_CH = 512
_NBUF = 4


def _patch_mm(x_hbm, w_ref, b_ref, acts_ref, o_hbm,
              xbuf, obuf, wc, insem, outsem, *, nch, chunks_per_batch):
    wc[...] = w_ref[...].astype(jnp.bfloat16)

    for s in range(_NBUF):
        pltpu.make_async_copy(
            x_hbm.at[pl.ds(s * _CH, _CH), :], xbuf.at[s], insem.at[s]
        ).start()

    def step(i, carry):
        s = jax.lax.rem(i, _NBUF)
        pltpu.make_async_copy(
            x_hbm.at[pl.ds(i * _CH, _CH), :], xbuf.at[s], insem.at[s]
        ).wait()
        y = jnp.dot(
            xbuf[s].astype(jnp.bfloat16), wc[...],
            preferred_element_type=jnp.float32,
        ) + b_ref[...]

        @pl.when(i >= _NBUF)
        def _():
            pltpu.make_async_copy(
                obuf.at[s], o_hbm.at[pl.ds((i - _NBUF) * _CH, _CH), :],
                outsem.at[s],
            ).wait()

        obuf[s] = y

        @pl.when(jax.lax.rem(i, chunks_per_batch) == 0)
        def _():
            obuf[s, _MASK_IDX, :] = acts_ref[0]

        pltpu.make_async_copy(
            obuf.at[s], o_hbm.at[pl.ds(i * _CH, _CH), :], outsem.at[s]
        ).start()

        @pl.when(i + _NBUF < nch)
        def _():
            pltpu.make_async_copy(
                x_hbm.at[pl.ds((i + _NBUF) * _CH, _CH), :], xbuf.at[s],
                insem.at[s],
            ).start()

        return carry

    jax.lax.fori_loop(0, nch, step, 0)

    for s in range(_NBUF):
        i = nch - _NBUF + s
        sl = jax.lax.rem(i, _NBUF)
        pltpu.make_async_copy(
            obuf.at[sl], o_hbm.at[pl.ds(i * _CH, _CH), :], outsem.at[sl]
        ).wait()


def kernel(x, W, b, acts):
    B, S, D = x.shape
    xf = x.reshape(B * S, D)
    b2 = b.reshape(1, D)
    acts2 = acts.reshape(1, D)
    nch = B * S // _CH
    out = pl.pallas_call(
        functools.partial(_patch_mm, nch=nch, chunks_per_batch=S // _CH),
        in_specs=[
            pl.BlockSpec(memory_space=pl.ANY),
            pl.BlockSpec((D, D), lambda: (0, 0)),
            pl.BlockSpec((1, D), lambda: (0, 0)),
            pl.BlockSpec((1, D), lambda: (0, 0)),
        ],
        out_specs=pl.BlockSpec(memory_space=pl.ANY),
        out_shape=jax.ShapeDtypeStruct((B * S, D), jnp.float32),
        scratch_shapes=[
            pltpu.VMEM((_NBUF, _CH, D), jnp.float32),
            pltpu.VMEM((_NBUF, _CH, D), jnp.float32),
            pltpu.VMEM((D, D), jnp.bfloat16),
            pltpu.SemaphoreType.DMA((_NBUF,)),
            pltpu.SemaphoreType.DMA((_NBUF,)),
        ],
    )(xf, W, b2, acts2)
    return out.reshape(B, S, D)
